# baseline (speedup 1.0000x reference)
import functools

import jax
import jax.numpy as jnp
from jax import lax
from jax.experimental import pallas as pl
from jax.experimental.pallas import tpu as pltpu
from jax.experimental.pallas import tpu_sc as plsc

NUM_OBJ = 5000
NUM_REL = 100000
NUM_OBJ_CLS = 151
NUM_REL_CLS = 51

BR = 1000
BO = 1000


def _xla_row_sum(e):
    n, c = e.shape
    cpad = ((c + 7) // 8) * 8
    if cpad != c:
        e = jnp.concatenate([e, jnp.zeros((n, cpad - c), e.dtype)], axis=1)
    t = e[:, 0:8]
    for k in range(1, cpad // 8):
        t = t + e[:, 8 * k:8 * k + 8]
    u = t[:, 0:4] + t[:, 4:8]
    v = u[:, 0:2] + u[:, 2:4]
    return v[:, 0:1] + v[:, 1:2]


def _softmax_stats(x, ncls):
    lane = jax.lax.broadcasted_iota(jnp.int32, x.shape, 1)
    valid = lane < ncls
    xm = jnp.where(valid, x, -jnp.inf)
    m = jnp.max(xm, axis=1, keepdims=True)
    e = jnp.exp(x - m)
    denom = _xla_row_sum(jnp.where(valid, e, 0.0))
    probs = e / denom
    probs_m = jnp.where(valid & (lane >= 1), probs, -1.0)
    score = jnp.max(probs_m, axis=1, keepdims=True)
    eq = probs_m == score
    cls = jnp.min(jnp.where(eq, lane, ncls + 1), axis=1, keepdims=True)
    return probs, score, cls


def _rel_body(x_ref, pair_ref, all_ref, score_ref):
    probs, score, cls = _softmax_stats(x_ref[...], NUM_REL_CLS)
    all_ref[:, pl.ds(0, NUM_REL_CLS)] = probs
    all_ref[:, pl.ds(NUM_REL_CLS, 1)] = lax.bitcast_convert_type(cls, jnp.float32)
    all_ref[:, pl.ds(NUM_REL_CLS + 1, 2)] = lax.bitcast_convert_type(
        pair_ref[...], jnp.float32)
    score_ref[...] = score


def _obj_body(x_ref, score_ref, cls_ref):
    _, score, cls = _softmax_stats(x_ref[...], NUM_OBJ_CLS)
    score_ref[...] = score
    cls_ref[...] = cls


@jax.jit
def _rel_stage(relation_logits, rel_pair_idx):
    grid = (NUM_REL // BR,)
    return pl.pallas_call(
        _rel_body,
        grid=grid,
        in_specs=[
            pl.BlockSpec((BR, NUM_REL_CLS), lambda i: (i, 0)),
            pl.BlockSpec((BR, 2), lambda i: (i, 0)),
        ],
        out_specs=[
            pl.BlockSpec((BR, 56), lambda i: (i, 0)),
            pl.BlockSpec((BR, 1), lambda i: (i, 0)),
        ],
        out_shape=[
            jax.ShapeDtypeStruct((NUM_REL, 56), jnp.float32),
            jax.ShapeDtypeStruct((NUM_REL, 1), jnp.float32),
        ],
    )(relation_logits, rel_pair_idx)


@jax.jit
def _obj_stage(refine_logits):
    grid = (NUM_OBJ // BO,)
    return pl.pallas_call(
        _obj_body,
        grid=grid,
        in_specs=[pl.BlockSpec((BO, NUM_OBJ_CLS), lambda i: (i, 0))],
        out_specs=[
            pl.BlockSpec((BO, 1), lambda i: (i, 0)),
            pl.BlockSpec((BO, 1), lambda i: (i, 0)),
        ],
        out_shape=[
            jax.ShapeDtypeStruct((NUM_OBJ, 1), jnp.float32),
            jax.ShapeDtypeStruct((NUM_OBJ, 1), jnp.int32),
        ],
    )(refine_logits)


_CH = 128
_NCH_FULL = 49
_NCH_LAST = 46
_ROWS_PER_TILE = _NCH_FULL * _CH
_TAIL = NUM_REL - (15 * _ROWS_PER_TILE + _NCH_LAST * _CH)
_TAIL_BASE = NUM_REL - _TAIL


def _sc_gather(order, allmat):
    npad = 16 * _ROWS_PER_TILE - NUM_REL
    order_main = jnp.concatenate(
        [order, jnp.zeros((npad,), jnp.int32)]).reshape(16, _NCH_FULL, _CH)
    order_tail = order[_TAIL_BASE:].reshape(1, _TAIL)
    mesh = plsc.VectorSubcoreMesh(core_axis_name="c", subcore_axis_name="s")

    @functools.partial(
        pl.kernel,
        mesh=mesh,
        compiler_params=pltpu.CompilerParams(use_tc_tiling_on_sc=False),
        out_type=jax.ShapeDtypeStruct((NUM_REL, 56), jnp.float32),
        scratch_types=[
            pltpu.VMEM((_NCH_FULL, _CH), jnp.int32),
            pltpu.VMEM((_CH, 56), jnp.float32),
            pltpu.VMEM((_TAIL, 56), jnp.float32),
            pltpu.VMEM((1, _TAIL), jnp.int32),
            pltpu.SemaphoreType.DMA,
        ],
    )
    def k(order_hbm, otail_hbm, all_hbm, oall, ordv, pbuf, tp, tord, sem):
        cid = lax.axis_index("c")
        sid = lax.axis_index("s")

        @pl.when(cid == 0)
        def _():
            pltpu.sync_copy(order_hbm.at[sid], ordv)
            nch = jnp.where(sid == 15, _NCH_LAST, _NCH_FULL)

            def chunk(cc, _):
                base = sid * _ROWS_PER_TILE + cc * _CH
                row = ordv.at[cc]
                pltpu.async_copy(all_hbm.at[row], pbuf, sem).wait()
                pltpu.sync_copy(pbuf, oall.at[pl.ds(base, _CH)])
                return _

            lax.fori_loop(0, nch, chunk, 0)

            @pl.when(sid == 15)
            def _tail():
                pltpu.sync_copy(otail_hbm, tord)
                row = tord.at[0]
                pltpu.async_copy(all_hbm.at[row], tp, sem).wait()
                pltpu.sync_copy(tp, oall.at[pl.ds(_TAIL_BASE, _TAIL)])

    return k(order_main, order_tail, allmat)


def kernel(relation_logits, refine_logits, rel_pair_idx, boxes):
    allmat, rel_score = _rel_stage(relation_logits, rel_pair_idx)
    obj_score, obj_cls = _obj_stage(refine_logits)
    rel_score = rel_score.reshape(-1)
    obj_scores = obj_score.reshape(-1)
    obj_class = obj_cls.reshape(-1)

    s0 = obj_scores[rel_pair_idx[:, 0]]
    s1 = obj_scores[rel_pair_idx[:, 1]]
    triple = rel_score * s0 * s1
    order = jnp.argsort(-triple)
    oall = _sc_gather(order, allmat)
    rel_probs_sorted = oall[:, :NUM_REL_CLS]
    rel_labels = lax.bitcast_convert_type(oall[:, NUM_REL_CLS], jnp.int32)
    rel_pair_sorted = lax.bitcast_convert_type(
        oall[:, NUM_REL_CLS + 1:NUM_REL_CLS + 3], jnp.int32)
    return (boxes, obj_class, obj_scores, rel_pair_sorted, rel_probs_sorted,
            rel_labels)


# speedup vs baseline: 2.2002x; 2.2002x over previous
import functools

import jax
import jax.numpy as jnp
from jax import lax
from jax.experimental import pallas as pl
from jax.experimental.pallas import tpu as pltpu
from jax.experimental.pallas import tpu_sc as plsc

NUM_OBJ = 5000
NUM_REL = 100000
NUM_OBJ_CLS = 151
NUM_REL_CLS = 51

BR = 1000
BO = 1000


def _xla_row_sum(e):
    n, c = e.shape
    cpad = ((c + 7) // 8) * 8
    if cpad != c:
        e = jnp.concatenate([e, jnp.zeros((n, cpad - c), e.dtype)], axis=1)
    t = e[:, 0:8]
    for k in range(1, cpad // 8):
        t = t + e[:, 8 * k:8 * k + 8]
    u = t[:, 0:4] + t[:, 4:8]
    v = u[:, 0:2] + u[:, 2:4]
    return v[:, 0:1] + v[:, 1:2]


def _softmax_stats(x, ncls):
    lane = jax.lax.broadcasted_iota(jnp.int32, x.shape, 1)
    valid = lane < ncls
    xm = jnp.where(valid, x, -jnp.inf)
    m = jnp.max(xm, axis=1, keepdims=True)
    e = jnp.exp(x - m)
    denom = _xla_row_sum(jnp.where(valid, e, 0.0))
    probs = e / denom
    probs_m = jnp.where(valid & (lane >= 1), probs, -1.0)
    score = jnp.max(probs_m, axis=1, keepdims=True)
    eq = probs_m == score
    cls = jnp.min(jnp.where(eq, lane, ncls + 1), axis=1, keepdims=True)
    return probs, score, cls


def _rel_body(x_ref, pair_ref, all_ref, score_ref):
    probs, score, cls = _softmax_stats(x_ref[...], NUM_REL_CLS)
    all_ref[:, pl.ds(0, NUM_REL_CLS)] = probs
    all_ref[:, pl.ds(NUM_REL_CLS, 1)] = lax.bitcast_convert_type(cls, jnp.float32)
    all_ref[:, pl.ds(NUM_REL_CLS + 1, 2)] = lax.bitcast_convert_type(
        pair_ref[...], jnp.float32)
    score_ref[...] = score


def _obj_body(x_ref, score_ref, cls_ref):
    _, score, cls = _softmax_stats(x_ref[...], NUM_OBJ_CLS)
    score_ref[...] = score
    cls_ref[...] = cls


@jax.jit
def _rel_stage(relation_logits, rel_pair_idx):
    grid = (NUM_REL // BR,)
    return pl.pallas_call(
        _rel_body,
        grid=grid,
        in_specs=[
            pl.BlockSpec((BR, NUM_REL_CLS), lambda i: (i, 0)),
            pl.BlockSpec((BR, 2), lambda i: (i, 0)),
        ],
        out_specs=[
            pl.BlockSpec((BR, 56), lambda i: (i, 0)),
            pl.BlockSpec((BR, 1), lambda i: (i, 0)),
        ],
        out_shape=[
            jax.ShapeDtypeStruct((NUM_REL, 56), jnp.float32),
            jax.ShapeDtypeStruct((NUM_REL, 1), jnp.float32),
        ],
    )(relation_logits, rel_pair_idx)


@jax.jit
def _obj_stage(refine_logits):
    grid = (NUM_OBJ // BO,)
    return pl.pallas_call(
        _obj_body,
        grid=grid,
        in_specs=[pl.BlockSpec((BO, NUM_OBJ_CLS), lambda i: (i, 0))],
        out_specs=[
            pl.BlockSpec((BO, 1), lambda i: (i, 0)),
            pl.BlockSpec((BO, 1), lambda i: (i, 0)),
        ],
        out_shape=[
            jax.ShapeDtypeStruct((NUM_OBJ, 1), jnp.float32),
            jax.ShapeDtypeStruct((NUM_OBJ, 1), jnp.int32),
        ],
    )(refine_logits)


_CH = 128
_NCH_FULL = 49
_NCH_LAST = 46
_ROWS_PER_TILE = _NCH_FULL * _CH
_TAIL = NUM_REL - (15 * _ROWS_PER_TILE + _NCH_LAST * _CH)
_TAIL_BASE = NUM_REL - _TAIL
_NSTAR = 100352
_PER_LANE = 392
_HALF = _ROWS_PER_TILE // 2


def _sc_sort_gather(sortin, objtbl, allmat):
    mesh = plsc.VectorSubcoreMesh(core_axis_name="c", subcore_axis_name="s")

    @functools.partial(
        pl.kernel,
        mesh=mesh,
        compiler_params=pltpu.CompilerParams(use_tc_tiling_on_sc=False,
                                             needs_layout_passes=False),
        out_type=jax.ShapeDtypeStruct((NUM_REL, 56), jnp.float32),
        scratch_types=[
            pltpu.VMEM((_HALF * 8,), jnp.float32),
            pltpu.VMEM((5056,), jnp.float32),
            pltpu.VMEM((_ROWS_PER_TILE,), jnp.int32),
            pltpu.VMEM((_ROWS_PER_TILE,), jnp.int32),
            pltpu.VMEM((_ROWS_PER_TILE,), jnp.int32),
            pltpu.VMEM((_ROWS_PER_TILE,), jnp.int32),
            pltpu.VMEM((_ROWS_PER_TILE,), jnp.int32),
            pltpu.VMEM((_NCH_FULL, _CH), jnp.int32),
            pltpu.VMEM((4096,), jnp.int32),
            pltpu.VMEM((4096,), jnp.int32),
            pltpu.VMEM((4096,), jnp.int32),
            pltpu.VMEM((4096,), jnp.int32),
            pltpu.VMEM((32, 128), jnp.int32),
            pltpu.VMEM((256,), jnp.int32),
            pltpu.VMEM((16,), jnp.int32),
            pltpu.VMEM((_CH, 56), jnp.float32),
            pltpu.VMEM((1, _TAIL), jnp.int32),
            pltpu.VMEM((_TAIL, 56), jnp.float32),
            pltpu.SemaphoreType.DMA,
            pltpu.SemaphoreType.DMA,
            pltpu.VMEM_SHARED((_NSTAR,), jnp.int32),
            pltpu.VMEM_SHARED((_NSTAR,), jnp.int32),
            pltpu.VMEM_SHARED((_NSTAR,), jnp.int32),
            pltpu.VMEM_SHARED((_NSTAR,), jnp.int32),
            pltpu.VMEM_SHARED((_NSTAR,), jnp.int32),
            pltpu.VMEM_SHARED((256,), jnp.int32),
        ],
    )
    def k(sortin_hbm, obj_hbm, all_hbm, oall,
          sinv, tblv, keyv, valv, kout, vout, posflat, posv2d,
          hist, offs, scanblk, idx1d, idx2d, tgall, tgv, pbuf, tord, tpbuf,
          sem, sem2, KeyA, ValA, KeyB, ValB, GH, TG2):
        cid = lax.axis_index("c")
        sid = lax.axis_index("s")

        @pl.when(cid == 0)
        def _():
            t = sid
            iota = lax.iota(jnp.int32, 16)
            zero16 = jnp.zeros((16,), jnp.int32)
            one16 = zero16 + 1

            def mkidx(i, c):
                j = 16 * i + iota
                idx1d[pl.ds(16 * i, 16)] = (
                    lax.shift_right_logical(j, 4) * 256 + 16 * t + (j & 15))
                return c
            lax.fori_loop(0, 256, mkidx, 0)
            pltpu.sync_copy(idx1d, GH.at[pl.ds(4096 * t, 4096)])
            for c in range(32):
                pltpu.sync_copy(GH.at[pl.ds(4096 * t + 128 * c, 128)],
                                idx2d.at[c])

            pltpu.sync_copy(obj_hbm, tblv)
            for half in range(2):
                pltpu.sync_copy(
                    sortin_hbm.at[pl.ds((t * _ROWS_PER_TILE + half * _HALF) * 8,
                                        _HALF * 8)],
                    sinv)

                def keygen(i, c):
                    j16 = 16 * i + iota
                    j8 = j16 * 8
                    r = plsc.load_gather(sinv, [j8])
                    p0 = plsc.load_gather(sinv, [j8 + 1]).astype(jnp.int32)
                    p1 = plsc.load_gather(sinv, [j8 + 2]).astype(jnp.int32)
                    s0 = plsc.load_gather(tblv, [p0])
                    s1 = plsc.load_gather(tblv, [p1])
                    key = plsc.bitcast((r * s0) * s1, jnp.int32)
                    off = half * _HALF + 16 * i
                    keyv[pl.ds(off, 16)] = key ^ (-1)
                    valv[pl.ds(off, 16)] = t * _ROWS_PER_TILE + half * _HALF + j16
                    return c
                lax.fori_loop(0, _HALF // 16, keygen, 0)
            pltpu.sync_copy(keyv, KeyA.at[pl.ds(t * _ROWS_PER_TILE, _ROWS_PER_TILE)])
            pltpu.sync_copy(valv, ValA.at[pl.ds(t * _ROWS_PER_TILE, _ROWS_PER_TILE)])
            plsc.subcore_barrier()

            for p in range(4):
                srcK, srcV, dstK, dstV = (
                    (KeyA, ValA, KeyB, ValB) if p % 2 == 0
                    else (KeyB, ValB, KeyA, ValA))
                sh = 8 * p
                if p > 0:
                    pltpu.sync_copy(
                        srcK.at[pl.ds(t * _ROWS_PER_TILE, _ROWS_PER_TILE)], keyv)
                    pltpu.sync_copy(
                        srcV.at[pl.ds(t * _ROWS_PER_TILE, _ROWS_PER_TILE)], valv)

                def zh(i, c):
                    hist[pl.ds(16 * i, 16)] = zero16
                    return c
                lax.fori_loop(0, 256, zh, 0)

                def hg(i, c):
                    kk = plsc.load_gather(keyv, [iota * _PER_LANE + i])
                    d = lax.shift_right_logical(kk, sh) & 255
                    plsc.addupdate_scatter(hist, [d * 16 + iota], one16)
                    return c
                lax.fori_loop(0, _PER_LANE, hg, 0)

                hs = [pltpu.async_copy(hist.at[pl.ds(128 * c, 128)],
                                       GH.at[idx2d.at[c]], sem2)
                      for c in range(32)]
                for h in hs:
                    h.wait()
                plsc.subcore_barrier()

                pltpu.sync_copy(GH.at[pl.ds(4096 * t, 4096)], scanblk)

                def scan_step(v, carry):
                    x = scanblk[pl.ds(16 * v, 16)]
                    cs = plsc.cumsum(x)
                    scanblk[pl.ds(16 * v, 16)] = cs - x + carry
                    return carry + jnp.sum(x)
                carry = lax.fori_loop(0, 256, scan_step, jnp.int32(0))
                tgv[...] = jnp.zeros((16,), jnp.int32) + carry
                pltpu.sync_copy(tgv, TG2.at[pl.ds(16 * t, 16)])
                plsc.subcore_barrier()

                pltpu.sync_copy(TG2, tgall)
                totv = plsc.load_gather(tgall, [iota * 16])
                base = jnp.sum(jnp.where(iota == t, plsc.cumsum(totv) - totv, 0))

                def ab(v, c):
                    scanblk[pl.ds(16 * v, 16)] = scanblk[pl.ds(16 * v, 16)] + base
                    return c
                lax.fori_loop(0, 256, ab, 0)
                pltpu.sync_copy(scanblk, GH.at[pl.ds(4096 * t, 4096)])
                plsc.subcore_barrier()

                hs = [pltpu.async_copy(GH.at[idx2d.at[c]],
                                       offs.at[pl.ds(128 * c, 128)], sem2)
                      for c in range(32)]
                for h in hs:
                    h.wait()
                plsc.subcore_barrier()

                def pm(i, c):
                    eidx = iota * _PER_LANE + i
                    kk = plsc.load_gather(keyv, [eidx])
                    vv = plsc.load_gather(valv, [eidx])
                    d = lax.shift_right_logical(kk, sh) & 255
                    oidx = d * 16 + iota
                    pos = plsc.load_gather(offs, [oidx])
                    plsc.store_scatter(offs, [oidx], pos + 1)
                    kout[pl.ds(16 * i, 16)] = kk
                    vout[pl.ds(16 * i, 16)] = vv
                    posflat[pl.ds(16 * i, 16)] = pos
                    return c
                lax.fori_loop(0, _PER_LANE, pm, 0)

                pltpu.sync_copy(
                    posflat, GH.at[pl.ds(t * _ROWS_PER_TILE, _ROWS_PER_TILE)])
                for c in range(_NCH_FULL):
                    pltpu.sync_copy(
                        GH.at[pl.ds(t * _ROWS_PER_TILE + 128 * c, 128)],
                        posv2d.at[c])
                hs = []
                for c in range(_NCH_FULL):
                    hs.append(pltpu.async_copy(kout.at[pl.ds(128 * c, 128)],
                                               dstK.at[posv2d.at[c]], sem2))
                    hs.append(pltpu.async_copy(vout.at[pl.ds(128 * c, 128)],
                                               dstV.at[posv2d.at[c]], sem2))
                for h in hs:
                    h.wait()
                plsc.subcore_barrier()

            nch = jnp.where(sid == 15, _NCH_LAST, _NCH_FULL)

            def chunk(cc, c):
                base = sid * _ROWS_PER_TILE + cc * _CH
                pltpu.sync_copy(ValA.at[pl.ds(base, _CH)], posv2d.at[cc])
                row = posv2d.at[cc]
                pltpu.async_copy(all_hbm.at[row], pbuf, sem).wait()
                pltpu.sync_copy(pbuf, oall.at[pl.ds(base, _CH)])
                return c
            lax.fori_loop(0, nch, chunk, 0)

            @pl.when(sid == 15)
            def _tail():
                pltpu.sync_copy(ValA.at[pl.ds(_TAIL_BASE, _TAIL)], tord.at[0])
                row = tord.at[0]
                pltpu.async_copy(all_hbm.at[row], tpbuf, sem).wait()
                pltpu.sync_copy(tpbuf, oall.at[pl.ds(_TAIL_BASE, _TAIL)])

    return k(sortin, objtbl, allmat)


def kernel(relation_logits, refine_logits, rel_pair_idx, boxes):
    allmat, rel_score = _rel_stage(relation_logits, rel_pair_idx)
    obj_score, obj_cls = _obj_stage(refine_logits)
    obj_scores = obj_score.reshape(-1)
    obj_class = obj_cls.reshape(-1)

    npad = _NSTAR - NUM_REL
    sortin = jnp.concatenate([
        jnp.pad(rel_score, ((0, npad), (0, 0))),
        jnp.pad(rel_pair_idx, ((0, npad), (0, 0))).astype(jnp.float32),
        jnp.zeros((_NSTAR, 5), jnp.float32)], axis=1)
    objtbl = jnp.pad(obj_scores, (0, 5056 - NUM_OBJ))

    oall = _sc_sort_gather(sortin.reshape(-1), objtbl, allmat)
    rel_probs_sorted = oall[:, :NUM_REL_CLS]
    rel_labels = lax.bitcast_convert_type(oall[:, NUM_REL_CLS], jnp.int32)
    rel_pair_sorted = lax.bitcast_convert_type(
        oall[:, NUM_REL_CLS + 1:NUM_REL_CLS + 3], jnp.int32)
    return (boxes, obj_class, obj_scores, rel_pair_sorted, rel_probs_sorted,
            rel_labels)


# speedup vs baseline: 2.2517x; 1.0234x over previous
import functools

import jax
import jax.numpy as jnp
from jax import lax
from jax.experimental import pallas as pl
from jax.experimental.pallas import tpu as pltpu
from jax.experimental.pallas import tpu_sc as plsc

NUM_OBJ = 5000
NUM_REL = 100000
NUM_OBJ_CLS = 151
NUM_REL_CLS = 51

BR = 1000
BO = 1000


def _xla_row_sum(e):
    n, c = e.shape
    cpad = ((c + 7) // 8) * 8
    if cpad != c:
        e = jnp.concatenate([e, jnp.zeros((n, cpad - c), e.dtype)], axis=1)
    t = e[:, 0:8]
    for k in range(1, cpad // 8):
        t = t + e[:, 8 * k:8 * k + 8]
    u = t[:, 0:4] + t[:, 4:8]
    v = u[:, 0:2] + u[:, 2:4]
    return v[:, 0:1] + v[:, 1:2]


def _softmax_stats(x, ncls):
    lane = jax.lax.broadcasted_iota(jnp.int32, x.shape, 1)
    valid = lane < ncls
    xm = jnp.where(valid, x, -jnp.inf)
    m = jnp.max(xm, axis=1, keepdims=True)
    e = jnp.exp(x - m)
    denom = _xla_row_sum(jnp.where(valid, e, 0.0))
    probs = e / denom
    probs_m = jnp.where(valid & (lane >= 1), probs, -1.0)
    score = jnp.max(probs_m, axis=1, keepdims=True)
    eq = probs_m == score
    cls = jnp.min(jnp.where(eq, lane, ncls + 1), axis=1, keepdims=True)
    return probs, score, cls


def _rel_body(x_ref, pair_ref, all_ref, score_ref):
    probs, score, cls = _softmax_stats(x_ref[...], NUM_REL_CLS)
    all_ref[:, pl.ds(0, NUM_REL_CLS)] = probs
    all_ref[:, pl.ds(NUM_REL_CLS, 1)] = lax.bitcast_convert_type(cls, jnp.float32)
    all_ref[:, pl.ds(NUM_REL_CLS + 1, 2)] = lax.bitcast_convert_type(
        pair_ref[...], jnp.float32)
    score_ref[...] = score


def _obj_body(x_ref, score_ref, cls_ref):
    _, score, cls = _softmax_stats(x_ref[...], NUM_OBJ_CLS)
    score_ref[...] = score
    cls_ref[...] = cls


@jax.jit
def _rel_stage(relation_logits, rel_pair_idx):
    grid = (NUM_REL // BR,)
    return pl.pallas_call(
        _rel_body,
        grid=grid,
        in_specs=[
            pl.BlockSpec((BR, NUM_REL_CLS), lambda i: (i, 0)),
            pl.BlockSpec((BR, 2), lambda i: (i, 0)),
        ],
        out_specs=[
            pl.BlockSpec((BR, 56), lambda i: (i, 0)),
            pl.BlockSpec((BR, 1), lambda i: (i, 0)),
        ],
        out_shape=[
            jax.ShapeDtypeStruct((NUM_REL, 56), jnp.float32),
            jax.ShapeDtypeStruct((NUM_REL, 1), jnp.float32),
        ],
    )(relation_logits, rel_pair_idx)


@jax.jit
def _obj_stage(refine_logits):
    grid = (NUM_OBJ // BO,)
    return pl.pallas_call(
        _obj_body,
        grid=grid,
        in_specs=[pl.BlockSpec((BO, NUM_OBJ_CLS), lambda i: (i, 0))],
        out_specs=[
            pl.BlockSpec((BO, 1), lambda i: (i, 0)),
            pl.BlockSpec((BO, 1), lambda i: (i, 0)),
        ],
        out_shape=[
            jax.ShapeDtypeStruct((NUM_OBJ, 1), jnp.float32),
            jax.ShapeDtypeStruct((NUM_OBJ, 1), jnp.int32),
        ],
    )(refine_logits)


_CH = 128
_NCH_FULL = 49
_NCH_LAST = 46
_ROWS_PER_TILE = _NCH_FULL * _CH
_TAIL = NUM_REL - (15 * _ROWS_PER_TILE + _NCH_LAST * _CH)
_TAIL_BASE = NUM_REL - _TAIL
_NSTAR = 100352
_PER_LANE = 392
_HALF = _ROWS_PER_TILE // 2
_QUARTER = _ROWS_PER_TILE // 4


def _sc_sort_gather(sortin, objtbl, allmat):
    mesh = plsc.VectorSubcoreMesh(core_axis_name="c", subcore_axis_name="s")

    @functools.partial(
        pl.kernel,
        mesh=mesh,
        compiler_params=pltpu.CompilerParams(use_tc_tiling_on_sc=False,
                                             needs_layout_passes=False),
        out_type=jax.ShapeDtypeStruct((NUM_REL, 56), jnp.float32),
        scratch_types=[
            pltpu.VMEM((_QUARTER * 8,), jnp.float32),
            pltpu.VMEM((5056,), jnp.float32),
            pltpu.VMEM((_ROWS_PER_TILE,), jnp.int32),
            pltpu.VMEM((_ROWS_PER_TILE,), jnp.int32),
            pltpu.VMEM((_ROWS_PER_TILE,), jnp.int32),
            pltpu.VMEM((_ROWS_PER_TILE,), jnp.int32),
            pltpu.VMEM((_ROWS_PER_TILE,), jnp.int32),
            pltpu.VMEM((_NCH_FULL, _CH), jnp.int32),
            pltpu.VMEM((4096,), jnp.int32),
            pltpu.VMEM((4096,), jnp.int32),
            pltpu.VMEM((4096,), jnp.int32),
            pltpu.VMEM((4096,), jnp.int32),
            pltpu.VMEM((32, 128), jnp.int32),
            pltpu.VMEM((256,), jnp.int32),
            pltpu.VMEM((16,), jnp.int32),
            pltpu.VMEM((2 * _CH, 56), jnp.float32),
            pltpu.VMEM((1, _TAIL), jnp.int32),
            pltpu.VMEM((_TAIL, 56), jnp.float32),
            pltpu.SemaphoreType.DMA,
            pltpu.SemaphoreType.DMA,
            pltpu.SemaphoreType.DMA,
            pltpu.VMEM_SHARED((_NSTAR,), jnp.int32),
            pltpu.VMEM_SHARED((_NSTAR,), jnp.int32),
            pltpu.VMEM_SHARED((_NSTAR,), jnp.int32),
            pltpu.VMEM_SHARED((_NSTAR,), jnp.int32),
            pltpu.VMEM_SHARED((_NSTAR,), jnp.int32),
            pltpu.VMEM_SHARED((256,), jnp.int32),
        ],
    )
    def k(sortin_hbm, obj_hbm, all_hbm, oall,
          sinv, tblv, keyv, valv, kout, vout, posflat, posv2d,
          hist, offs, scanblk, idx1d, idx2d, tgall, tgv, pbuf, tord, tpbuf,
          sem, sem2, semw, KeyA, ValA, KeyB, ValB, GH, TG2):
        cid = lax.axis_index("c")
        sid = lax.axis_index("s")

        @pl.when(cid == 0)
        def _():
            t = sid
            iota = lax.iota(jnp.int32, 16)
            zero16 = jnp.zeros((16,), jnp.int32)
            one16 = zero16 + 1

            def mkidx(i, c):
                j = 16 * i + iota
                idx1d[pl.ds(16 * i, 16)] = (
                    lax.shift_right_logical(j, 4) * 256 + 16 * t + (j & 15))
                return c
            lax.fori_loop(0, 256, mkidx, 0)
            pltpu.sync_copy(idx1d, GH.at[pl.ds(4096 * t, 4096)])
            for c in range(32):
                pltpu.sync_copy(GH.at[pl.ds(4096 * t + 128 * c, 128)],
                                idx2d.at[c])

            pltpu.sync_copy(obj_hbm, tblv)
            for half in range(4):
                pltpu.sync_copy(
                    sortin_hbm.at[pl.ds((t * _ROWS_PER_TILE + half * _QUARTER) * 8,
                                        _QUARTER * 8)],
                    sinv)

                def keygen(i, c):
                    j16 = 16 * i + iota
                    j8 = j16 * 8
                    r = plsc.load_gather(sinv, [j8])
                    p0 = plsc.load_gather(sinv, [j8 + 1]).astype(jnp.int32)
                    p1 = plsc.load_gather(sinv, [j8 + 2]).astype(jnp.int32)
                    s0 = plsc.load_gather(tblv, [p0])
                    s1 = plsc.load_gather(tblv, [p1])
                    key = plsc.bitcast((r * s0) * s1, jnp.int32)
                    off = half * _QUARTER + 16 * i
                    keyv[pl.ds(off, 16)] = key ^ (-1)
                    valv[pl.ds(off, 16)] = t * _ROWS_PER_TILE + half * _QUARTER + j16
                    return c
                lax.fori_loop(0, _QUARTER // 16, keygen, 0)
            pltpu.sync_copy(keyv, KeyA.at[pl.ds(t * _ROWS_PER_TILE, _ROWS_PER_TILE)])
            pltpu.sync_copy(valv, ValA.at[pl.ds(t * _ROWS_PER_TILE, _ROWS_PER_TILE)])
            plsc.subcore_barrier()

            for p in range(4):
                srcK, srcV, dstK, dstV = (
                    (KeyA, ValA, KeyB, ValB) if p % 2 == 0
                    else (KeyB, ValB, KeyA, ValA))
                sh = 8 * p
                if p > 0:
                    pltpu.sync_copy(
                        srcK.at[pl.ds(t * _ROWS_PER_TILE, _ROWS_PER_TILE)], keyv)
                    pltpu.sync_copy(
                        srcV.at[pl.ds(t * _ROWS_PER_TILE, _ROWS_PER_TILE)], valv)

                def zh(i, c):
                    hist[pl.ds(16 * i, 16)] = zero16
                    return c
                lax.fori_loop(0, 256, zh, 0)

                def hg(i, c):
                    kk = plsc.load_gather(keyv, [iota * _PER_LANE + i])
                    d = lax.shift_right_logical(kk, sh) & 255
                    plsc.addupdate_scatter(hist, [d * 16 + iota], one16)
                    return c
                lax.fori_loop(0, _PER_LANE, hg, 0)

                hs = [pltpu.async_copy(hist.at[pl.ds(128 * c, 128)],
                                       GH.at[idx2d.at[c]], sem2)
                      for c in range(32)]
                for h in hs:
                    h.wait()
                plsc.subcore_barrier()

                pltpu.sync_copy(GH.at[pl.ds(4096 * t, 4096)], scanblk)

                def scan_step(v, carry):
                    x = scanblk[pl.ds(16 * v, 16)]
                    cs = plsc.cumsum(x)
                    scanblk[pl.ds(16 * v, 16)] = cs - x + carry
                    return carry + jnp.sum(x)
                carry = lax.fori_loop(0, 256, scan_step, jnp.int32(0))
                tgv[...] = jnp.zeros((16,), jnp.int32) + carry
                pltpu.sync_copy(tgv, TG2.at[pl.ds(16 * t, 16)])
                plsc.subcore_barrier()

                pltpu.sync_copy(TG2, tgall)
                totv = plsc.load_gather(tgall, [iota * 16])
                base = jnp.sum(jnp.where(iota == t, plsc.cumsum(totv) - totv, 0))

                def ab(v, c):
                    scanblk[pl.ds(16 * v, 16)] = scanblk[pl.ds(16 * v, 16)] + base
                    return c
                lax.fori_loop(0, 256, ab, 0)
                pltpu.sync_copy(scanblk, GH.at[pl.ds(4096 * t, 4096)])
                plsc.subcore_barrier()

                hs = [pltpu.async_copy(GH.at[idx2d.at[c]],
                                       offs.at[pl.ds(128 * c, 128)], sem2)
                      for c in range(32)]
                for h in hs:
                    h.wait()
                plsc.subcore_barrier()

                def pm(i, c):
                    eidx = iota * _PER_LANE + i
                    kk = plsc.load_gather(keyv, [eidx])
                    vv = plsc.load_gather(valv, [eidx])
                    d = lax.shift_right_logical(kk, sh) & 255
                    oidx = d * 16 + iota
                    pos = plsc.load_gather(offs, [oidx])
                    plsc.store_scatter(offs, [oidx], pos + 1)
                    kout[pl.ds(16 * i, 16)] = kk
                    vout[pl.ds(16 * i, 16)] = vv
                    posflat[pl.ds(16 * i, 16)] = pos
                    return c
                lax.fori_loop(0, _PER_LANE, pm, 0)

                pltpu.sync_copy(
                    posflat, GH.at[pl.ds(t * _ROWS_PER_TILE, _ROWS_PER_TILE)])
                for c in range(_NCH_FULL):
                    pltpu.sync_copy(
                        GH.at[pl.ds(t * _ROWS_PER_TILE + 128 * c, 128)],
                        posv2d.at[c])
                hs = []
                for c in range(_NCH_FULL):
                    hs.append(pltpu.async_copy(kout.at[pl.ds(128 * c, 128)],
                                               dstK.at[posv2d.at[c]], sem2))
                    hs.append(pltpu.async_copy(vout.at[pl.ds(128 * c, 128)],
                                               dstV.at[posv2d.at[c]], sem2))
                for h in hs:
                    h.wait()
                plsc.subcore_barrier()

            def chunk2(j, c):
                c0 = 2 * j
                base0 = sid * _ROWS_PER_TILE + c0 * _CH
                pltpu.sync_copy(ValA.at[pl.ds(base0, _CH)], posv2d.at[c0])
                pltpu.sync_copy(ValA.at[pl.ds(base0 + _CH, _CH)],
                                posv2d.at[c0 + 1])
                h0 = pltpu.async_copy(all_hbm.at[posv2d.at[c0]],
                                      pbuf.at[pl.ds(0, _CH)], sem)
                h1 = pltpu.async_copy(all_hbm.at[posv2d.at[c0 + 1]],
                                      pbuf.at[pl.ds(_CH, _CH)], semw)
                h0.wait()
                pltpu.sync_copy(pbuf.at[pl.ds(0, _CH)],
                                oall.at[pl.ds(base0, _CH)])
                h1.wait()
                pltpu.sync_copy(pbuf.at[pl.ds(_CH, _CH)],
                                oall.at[pl.ds(base0 + _CH, _CH)])
                return c

            npairs = jnp.where(sid == 15, _NCH_LAST // 2, _NCH_FULL // 2)
            lax.fori_loop(0, npairs, chunk2, 0)

            @pl.when(sid < 15)
            def _last():
                cc = _NCH_FULL - 1
                base = sid * _ROWS_PER_TILE + cc * _CH
                pltpu.sync_copy(ValA.at[pl.ds(base, _CH)], posv2d.at[cc])
                pltpu.async_copy(all_hbm.at[posv2d.at[cc]],
                                 pbuf.at[pl.ds(0, _CH)], sem).wait()
                pltpu.sync_copy(pbuf.at[pl.ds(0, _CH)],
                                oall.at[pl.ds(base, _CH)])

            @pl.when(sid == 15)
            def _tail():
                pltpu.sync_copy(ValA.at[pl.ds(_TAIL_BASE, _TAIL)], tord.at[0])
                row = tord.at[0]
                pltpu.async_copy(all_hbm.at[row], tpbuf, sem).wait()
                pltpu.sync_copy(tpbuf, oall.at[pl.ds(_TAIL_BASE, _TAIL)])

    return k(sortin, objtbl, allmat)


def kernel(relation_logits, refine_logits, rel_pair_idx, boxes):
    allmat, rel_score = _rel_stage(relation_logits, rel_pair_idx)
    obj_score, obj_cls = _obj_stage(refine_logits)
    obj_scores = obj_score.reshape(-1)
    obj_class = obj_cls.reshape(-1)

    npad = _NSTAR - NUM_REL
    sortin = jnp.concatenate([
        jnp.pad(rel_score, ((0, npad), (0, 0))),
        jnp.pad(rel_pair_idx, ((0, npad), (0, 0))).astype(jnp.float32),
        jnp.zeros((_NSTAR, 5), jnp.float32)], axis=1)
    objtbl = jnp.pad(obj_scores, (0, 5056 - NUM_OBJ))

    oall = _sc_sort_gather(sortin.reshape(-1), objtbl, allmat)
    rel_probs_sorted = oall[:, :NUM_REL_CLS]
    rel_labels = lax.bitcast_convert_type(oall[:, NUM_REL_CLS], jnp.int32)
    rel_pair_sorted = lax.bitcast_convert_type(
        oall[:, NUM_REL_CLS + 1:NUM_REL_CLS + 3], jnp.int32)
    return (boxes, obj_class, obj_scores, rel_pair_sorted, rel_probs_sorted,
            rel_labels)
